# baseline (speedup 1.0000x reference)
import functools
import math

import jax
import jax.numpy as jnp
from jax import lax
from jax.experimental import pallas as pl
from jax.experimental.pallas import tpu as pltpu
from jax.experimental.pallas import tpu_sc as plsc

_NUM_CAMS = 1000
_N_RAYS = 16384
_PC = 1024
_D = 16

_NTH = 10
_CA = [(-1.0) ** i / math.factorial(2 * i + 1) for i in range(_NTH + 1)]
_CB = [(-1.0) ** i / math.factorial(2 * i + 2) for i in range(_NTH + 1)]
_CC = [(-1.0) ** i / math.factorial(2 * i + 3) for i in range(_NTH + 1)]


def _horner(x2, coeffs):
    acc = coeffs[-1] * jnp.ones_like(x2)
    for c in reversed(coeffs[:-1]):
        acc = acc * x2 + c
    return acc


def _pose_table_tc(wu_ref, out_ref):
    w0 = wu_ref[0:1, :]
    w1 = wu_ref[1:2, :]
    w2 = wu_ref[2:3, :]
    u0 = wu_ref[3:4, :]
    u1 = wu_ref[4:5, :]
    u2 = wu_ref[5:6, :]
    s00 = w0 * w0
    s11 = w1 * w1
    s22 = w2 * w2
    x2 = s00 + s11 + s22
    A = _horner(x2, _CA)
    B = _horner(x2, _CB)
    C = _horner(x2, _CC)
    p01 = w0 * w1
    p02 = w0 * w2
    p12 = w1 * w2
    r00 = 1.0 - B * (s11 + s22)
    r01 = B * p01 - A * w2
    r02 = B * p02 + A * w1
    r10 = B * p01 + A * w2
    r11 = 1.0 - B * (s00 + s22)
    r12 = B * p12 - A * w0
    r20 = B * p02 - A * w1
    r21 = B * p12 + A * w0
    r22 = 1.0 - B * (s00 + s11)
    v00 = 1.0 - C * (s11 + s22)
    v01 = C * p01 - B * w2
    v02 = C * p02 + B * w1
    v10 = C * p01 + B * w2
    v11 = 1.0 - C * (s00 + s22)
    v12 = C * p12 - B * w0
    v20 = C * p02 - B * w1
    v21 = C * p12 + B * w0
    v22 = 1.0 - C * (s00 + s11)
    t0 = v00 * u0 + v01 * u1 + v02 * u2
    t1 = v10 * u0 + v11 * u1 + v12 * u2
    t2 = v20 * u0 + v21 * u1 + v22 * u2
    zero = jnp.zeros_like(w0)
    one = jnp.ones_like(w0)
    m = jnp.concatenate(
        [r00, r01, r02, t0,
         r10, r11, r12, t1,
         r20, r21, r22, t2,
         zero, zero, zero, one], axis=0)
    out_ref[...] = m.T


_NW = 32
_BPW = _N_RAYS // _NW


@functools.partial(
    pl.kernel,
    mesh=plsc.VectorSubcoreMesh(core_axis_name="c", subcore_axis_name="s"),
    out_type=jax.ShapeDtypeStruct((_N_RAYS, _D), jnp.float32),
    scratch_types=[
        pltpu.VMEM((_BPW,), jnp.int32),
        pltpu.VMEM((_BPW, _D), jnp.float32),
        pltpu.SemaphoreType.DMA,
    ],
    compiler_params=pltpu.CompilerParams(use_tc_tiling_on_sc=False),
)
def _gather_sc(table_hbm, idx_hbm, out_hbm, idx_v, rows_v, sem):
    wid = lax.axis_index("s") * 2 + lax.axis_index("c")
    base = wid * _BPW
    pltpu.sync_copy(idx_hbm.at[pl.ds(base, _BPW)], idx_v)
    pltpu.async_copy(table_hbm.at[idx_v], rows_v, sem).wait()
    pltpu.sync_copy(rows_v, out_hbm.at[pl.ds(base, _BPW)])


def kernel(r, t, cam_id):
    wu = jnp.concatenate([r, t], axis=1)
    wu = jnp.pad(wu, ((0, _PC - _NUM_CAMS), (0, 0)))
    wu_t = jnp.pad(wu.T, ((0, 2), (0, 0)))
    table = pl.pallas_call(
        _pose_table_tc,
        out_shape=jax.ShapeDtypeStruct((_PC, _D), jnp.float32),
    )(wu_t)
    out = _gather_sc(table, cam_id.astype(jnp.int32))
    return out.reshape(_N_RAYS, 4, 4)


# speedup vs baseline: 1.0017x; 1.0017x over previous
import functools
import math

import jax
import jax.numpy as jnp
from jax import lax
from jax.experimental import pallas as pl
from jax.experimental.pallas import tpu as pltpu
from jax.experimental.pallas import tpu_sc as plsc

_NUM_CAMS = 1000
_N_RAYS = 16384
_PC = 1024
_D = 16

_NTH = 10
_CA = [(-1.0) ** i / math.factorial(2 * i + 1) for i in range(_NTH + 1)]
_CB = [(-1.0) ** i / math.factorial(2 * i + 2) for i in range(_NTH + 1)]
_CC = [(-1.0) ** i / math.factorial(2 * i + 3) for i in range(_NTH + 1)]


def _horner(x2, coeffs):
    acc = coeffs[-1] * jnp.ones_like(x2)
    for c in reversed(coeffs[:-1]):
        acc = acc * x2 + c
    return acc


def _pose_table_tc(wu_ref, out_ref):
    w0 = wu_ref[0:1, :]
    w1 = wu_ref[1:2, :]
    w2 = wu_ref[2:3, :]
    u0 = wu_ref[3:4, :]
    u1 = wu_ref[4:5, :]
    u2 = wu_ref[5:6, :]
    s00 = w0 * w0
    s11 = w1 * w1
    s22 = w2 * w2
    x2 = s00 + s11 + s22
    A = _horner(x2, _CA)
    B = _horner(x2, _CB)
    C = _horner(x2, _CC)
    p01 = w0 * w1
    p02 = w0 * w2
    p12 = w1 * w2
    r00 = 1.0 - B * (s11 + s22)
    r01 = B * p01 - A * w2
    r02 = B * p02 + A * w1
    r10 = B * p01 + A * w2
    r11 = 1.0 - B * (s00 + s22)
    r12 = B * p12 - A * w0
    r20 = B * p02 - A * w1
    r21 = B * p12 + A * w0
    r22 = 1.0 - B * (s00 + s11)
    v00 = 1.0 - C * (s11 + s22)
    v01 = C * p01 - B * w2
    v02 = C * p02 + B * w1
    v10 = C * p01 + B * w2
    v11 = 1.0 - C * (s00 + s22)
    v12 = C * p12 - B * w0
    v20 = C * p02 - B * w1
    v21 = C * p12 + B * w0
    v22 = 1.0 - C * (s00 + s11)
    t0 = v00 * u0 + v01 * u1 + v02 * u2
    t1 = v10 * u0 + v11 * u1 + v12 * u2
    t2 = v20 * u0 + v21 * u1 + v22 * u2
    zero = jnp.zeros_like(w0)
    one = jnp.ones_like(w0)
    m = jnp.concatenate(
        [r00, r01, r02, t0,
         r10, r11, r12, t1,
         r20, r21, r22, t2,
         zero, zero, zero, one], axis=0)
    out_ref[...] = m.T


_NW = 32
_BPW = _N_RAYS // _NW


@functools.partial(
    pl.kernel,
    mesh=plsc.VectorSubcoreMesh(core_axis_name="c", subcore_axis_name="s"),
    out_type=jax.ShapeDtypeStruct((_N_RAYS, _D), jnp.float32),
    scratch_types=[
        pltpu.VMEM((_BPW,), jnp.int32),
        pltpu.VMEM((_BPW, _D), jnp.float32),
        pltpu.VMEM_SHARED((_PC, _D), jnp.float32),
        pltpu.SemaphoreType.DMA,
    ],
    compiler_params=pltpu.CompilerParams(use_tc_tiling_on_sc=False),
)
def _gather_sc(table_hbm, idx_hbm, out_hbm, idx_v, rows_v, table_sh, sem):
    sid = lax.axis_index("s")
    wid = sid * 2 + lax.axis_index("c")
    base = wid * _BPW
    @pl.when(sid == 0)
    def _():
        pltpu.sync_copy(table_hbm, table_sh)
    pltpu.sync_copy(idx_hbm.at[pl.ds(base, _BPW)], idx_v)
    plsc.subcore_barrier()
    pltpu.async_copy(table_sh.at[idx_v], rows_v, sem).wait()
    pltpu.sync_copy(rows_v, out_hbm.at[pl.ds(base, _BPW)])


def kernel(r, t, cam_id):
    wu = jnp.concatenate([r, t], axis=1)
    wu = jnp.pad(wu, ((0, _PC - _NUM_CAMS), (0, 0)))
    wu_t = jnp.pad(wu.T, ((0, 2), (0, 0)))
    table = pl.pallas_call(
        _pose_table_tc,
        out_shape=jax.ShapeDtypeStruct((_PC, _D), jnp.float32),
    )(wu_t)
    out = _gather_sc(table, cam_id.astype(jnp.int32))
    return out.reshape(_N_RAYS, 4, 4)


# speedup vs baseline: 1.2272x; 1.2251x over previous
import functools
import math

import jax
import jax.numpy as jnp
from jax import lax
from jax.experimental import pallas as pl
from jax.experimental.pallas import tpu as pltpu
from jax.experimental.pallas import tpu_sc as plsc

_NUM_CAMS = 1000
_N_RAYS = 16384
_PC = 1024
_D = 16

_NTH = 10
_CA = [(-1.0) ** i / math.factorial(2 * i + 1) for i in range(_NTH + 1)]
_CB = [(-1.0) ** i / math.factorial(2 * i + 2) for i in range(_NTH + 1)]
_CC = [(-1.0) ** i / math.factorial(2 * i + 3) for i in range(_NTH + 1)]

_NW = 32
_BPW = _N_RAYS // _NW
_CPS = _PC // 16
_NG = _CPS // 16
_GPW = _BPW // 16


def _horner(x2, coeffs):
    acc = coeffs[-1] * jnp.ones_like(x2)
    for c in reversed(coeffs[:-1]):
        acc = acc * x2 + c
    return acc


def _pose_rows(w0, w1, w2, u0, u1, u2):
    s00 = w0 * w0
    s11 = w1 * w1
    s22 = w2 * w2
    x2 = s00 + s11 + s22
    A = _horner(x2, _CA)
    B = _horner(x2, _CB)
    C = _horner(x2, _CC)
    p01 = w0 * w1
    p02 = w0 * w2
    p12 = w1 * w2
    r00 = 1.0 - B * (s11 + s22)
    r01 = B * p01 - A * w2
    r02 = B * p02 + A * w1
    r10 = B * p01 + A * w2
    r11 = 1.0 - B * (s00 + s22)
    r12 = B * p12 - A * w0
    r20 = B * p02 - A * w1
    r21 = B * p12 + A * w0
    r22 = 1.0 - B * (s00 + s11)
    v00 = 1.0 - C * (s11 + s22)
    v01 = C * p01 - B * w2
    v02 = C * p02 + B * w1
    v10 = C * p01 + B * w2
    v11 = 1.0 - C * (s00 + s22)
    v12 = C * p12 - B * w0
    v20 = C * p02 - B * w1
    v21 = C * p12 + B * w0
    v22 = 1.0 - C * (s00 + s11)
    t0 = v00 * u0 + v01 * u1 + v02 * u2
    t1 = v10 * u0 + v11 * u1 + v12 * u2
    t2 = v20 * u0 + v21 * u1 + v22 * u2
    zero = jnp.zeros_like(w0)
    one = zero + 1.0
    return [r00, r01, r02, t0,
            r10, r11, r12, t1,
            r20, r21, r22, t2,
            zero, zero, zero, one]


@functools.partial(
    pl.kernel,
    mesh=plsc.VectorSubcoreMesh(core_axis_name="c", subcore_axis_name="s"),
    out_type=jax.ShapeDtypeStruct((_N_RAYS * _D // 128, 128), jnp.float32),
    scratch_types=[
        pltpu.VMEM((6 * _PC,), jnp.float32),
        pltpu.VMEM((_BPW,), jnp.int32),
        pltpu.VMEM((_D, _PC), jnp.float32),
        pltpu.VMEM((8, 128), jnp.float32),
        pltpu.VMEM((4 * _D, 128), jnp.float32),
        pltpu.VMEM_SHARED((_D, _PC), jnp.float32),
        pltpu.SemaphoreType.DMA,
    ],
    compiler_params=pltpu.CompilerParams(needs_layout_passes=False),
)
def _pose_sc(wu_hbm, idx_hbm, out_hbm, wu_v, idx_v, tbl_v, stg_v, buf_v,
             tbl_sh, sem):
    sid = lax.axis_index("s")
    wid = sid * 2 + lax.axis_index("c")
    base = wid * _BPW
    pltpu.sync_copy(wu_hbm, wu_v)
    pltpu.sync_copy(idx_hbm.at[pl.ds(base, _BPW)], idx_v)
    chunk0 = (sid // 2) * 128
    e0 = (sid % 2) * 8
    for g in range(8):
        o = chunk0 + g * 16
        m = _pose_rows(wu_v[pl.ds(o, 16)],
                       wu_v[pl.ds(_PC + o, 16)],
                       wu_v[pl.ds(2 * _PC + o, 16)],
                       wu_v[pl.ds(3 * _PC + o, 16)],
                       wu_v[pl.ds(4 * _PC + o, 16)],
                       wu_v[pl.ds(5 * _PC + o, 16)])

        @pl.when(e0 == 0)
        def _():
            for k in range(8):
                stg_v[k, pl.ds(g * 16, 16)] = m[k]

        @pl.when(e0 == 8)
        def _():
            for k in range(8):
                stg_v[k, pl.ds(g * 16, 16)] = m[8 + k]

    pltpu.sync_copy(stg_v, tbl_sh.at[pl.ds(e0, 8), pl.ds(chunk0, 128)])
    plsc.subcore_barrier()
    pltpu.sync_copy(tbl_sh, tbl_v)
    for g in range(_GPW):
        cam = idx_v[pl.ds(g * 16, 16)]
        t, lo = g // 8, (g % 8) * 16
        for k in range(_D):
            i, j = k // 4, k % 4
            val = plsc.load_gather(tbl_v, [jnp.full((16,), k, jnp.int32),
                                           cam])
            buf_v[(i * 4 + t) * 4 + j, pl.ds(lo, 16)] = val
    t0 = wid * 16
    for i in range(4):
        pltpu.sync_copy(buf_v.at[pl.ds(i * 16, 16)],
                        out_hbm.at[pl.ds(i * 512 + t0, 16)])


def kernel(r, t, cam_id):
    wu = jnp.concatenate([r, t], axis=1)
    wu = jnp.pad(wu, ((0, _PC - _NUM_CAMS), (0, 0)))
    wu_flat = wu.T.reshape(6 * _PC)
    out2d = _pose_sc(wu_flat, cam_id.astype(jnp.int32))
    out = out2d.reshape(4, 128, 4, 128).transpose(1, 3, 0, 2)
    return out.reshape(_N_RAYS, 4, 4)


# speedup vs baseline: 1.2356x; 1.0069x over previous
import functools
import math

import jax
import jax.numpy as jnp
from jax import lax
from jax.experimental import pallas as pl
from jax.experimental.pallas import tpu as pltpu
from jax.experimental.pallas import tpu_sc as plsc

_NUM_CAMS = 1000
_N_RAYS = 16384
_PC = 1024
_D = 16

_NTH = 10
_CA = [(-1.0) ** i / math.factorial(2 * i + 1) for i in range(_NTH + 1)]
_CB = [(-1.0) ** i / math.factorial(2 * i + 2) for i in range(_NTH + 1)]
_CC = [(-1.0) ** i / math.factorial(2 * i + 3) for i in range(_NTH + 1)]

_NW = 32
_BPW = _N_RAYS // _NW
_CPS = _PC // 16
_NG = _CPS // 16
_GPW = _BPW // 16


def _horner(x2, coeffs):
    acc = coeffs[-1] * jnp.ones_like(x2)
    for c in reversed(coeffs[:-1]):
        acc = acc * x2 + c
    return acc


def _pose_rows(w0, w1, w2, u0, u1, u2):
    s00 = w0 * w0
    s11 = w1 * w1
    s22 = w2 * w2
    x2 = s00 + s11 + s22
    A = _horner(x2, _CA)
    B = _horner(x2, _CB)
    C = _horner(x2, _CC)
    p01 = w0 * w1
    p02 = w0 * w2
    p12 = w1 * w2
    r00 = 1.0 - B * (s11 + s22)
    r01 = B * p01 - A * w2
    r02 = B * p02 + A * w1
    r10 = B * p01 + A * w2
    r11 = 1.0 - B * (s00 + s22)
    r12 = B * p12 - A * w0
    r20 = B * p02 - A * w1
    r21 = B * p12 + A * w0
    r22 = 1.0 - B * (s00 + s11)
    v00 = 1.0 - C * (s11 + s22)
    v01 = C * p01 - B * w2
    v02 = C * p02 + B * w1
    v10 = C * p01 + B * w2
    v11 = 1.0 - C * (s00 + s22)
    v12 = C * p12 - B * w0
    v20 = C * p02 - B * w1
    v21 = C * p12 + B * w0
    v22 = 1.0 - C * (s00 + s11)
    t0 = v00 * u0 + v01 * u1 + v02 * u2
    t1 = v10 * u0 + v11 * u1 + v12 * u2
    t2 = v20 * u0 + v21 * u1 + v22 * u2
    zero = jnp.zeros_like(w0)
    one = zero + 1.0
    return [r00, r01, r02, t0,
            r10, r11, r12, t1,
            r20, r21, r22, t2,
            zero, zero, zero, one]


@functools.partial(
    pl.kernel,
    mesh=plsc.VectorSubcoreMesh(core_axis_name="c", subcore_axis_name="s"),
    out_type=jax.ShapeDtypeStruct((_N_RAYS * _D // 128, 128), jnp.float32),
    scratch_types=[
        pltpu.VMEM((6 * _PC,), jnp.float32),
        pltpu.VMEM((_BPW,), jnp.int32),
        pltpu.VMEM((_D, _PC), jnp.float32),
        pltpu.VMEM((8, 128), jnp.float32),
        pltpu.VMEM((4 * _D, 128), jnp.float32),
        pltpu.VMEM_SHARED((_D, _PC), jnp.float32),
        pltpu.SemaphoreType.DMA,
    ],
    compiler_params=pltpu.CompilerParams(needs_layout_passes=False),
)
def _pose_sc(wu_hbm, idx_hbm, out_hbm, wu_v, idx_v, tbl_v, stg_v, buf_v,
             tbl_sh, sem):
    sid = lax.axis_index("s")
    wid = sid * 2 + lax.axis_index("c")
    base = wid * _BPW
    pltpu.sync_copy(wu_hbm, wu_v)
    pltpu.sync_copy(idx_hbm.at[pl.ds(base, _BPW)], idx_v)
    chunk0 = (sid // 2) * 128
    e0 = (sid % 2) * 8
    for g in range(8):
        o = chunk0 + g * 16
        m = _pose_rows(wu_v[pl.ds(o, 16)],
                       wu_v[pl.ds(_PC + o, 16)],
                       wu_v[pl.ds(2 * _PC + o, 16)],
                       wu_v[pl.ds(3 * _PC + o, 16)],
                       wu_v[pl.ds(4 * _PC + o, 16)],
                       wu_v[pl.ds(5 * _PC + o, 16)])

        @pl.when(e0 == 0)
        def _():
            for k in range(8):
                stg_v[k, pl.ds(g * 16, 16)] = m[k]

        @pl.when(e0 == 8)
        def _():
            for k in range(8):
                stg_v[k, pl.ds(g * 16, 16)] = m[8 + k]

    pltpu.sync_copy(stg_v, tbl_sh.at[pl.ds(e0, 8), pl.ds(chunk0, 128)])
    plsc.subcore_barrier()
    pltpu.sync_copy(tbl_sh, tbl_v)
    t0 = wid * 16
    copies = []
    for i in range(4):
        for g in range(_GPW):
            cam = idx_v[pl.ds(g * 16, 16)]
            t, lo = g // 8, (g % 8) * 16
            for j in range(4):
                k = i * 4 + j
                val = plsc.load_gather(
                    tbl_v, [jnp.full((16,), k, jnp.int32), cam])
                buf_v[(i * 4 + t) * 4 + j, pl.ds(lo, 16)] = val
        copies.append(pltpu.async_copy(buf_v.at[pl.ds(i * 16, 16)],
                                       out_hbm.at[pl.ds(i * 512 + t0, 16)],
                                       sem))
    for c in copies:
        c.wait()


def kernel(r, t, cam_id):
    wu = jnp.concatenate([r, t], axis=1)
    wu = jnp.pad(wu, ((0, _PC - _NUM_CAMS), (0, 0)))
    wu_flat = wu.T.reshape(6 * _PC)
    out2d = _pose_sc(wu_flat, cam_id.astype(jnp.int32))
    out = out2d.reshape(4, 128, 4, 128).transpose(1, 3, 0, 2)
    return out.reshape(_N_RAYS, 4, 4)
